# baseline (speedup 1.0000x reference)
import functools

import jax
import jax.numpy as jnp
from jax import lax
from jax.experimental import pallas as pl
from jax.experimental.pallas import tpu as pltpu
from jax.experimental.pallas import tpu_sc as plsc

N_NODES = 10000
N_EDGES = 320000
D_FEAT = 128
D_HID = 16
N_CLASSES = 7

NC = 2
NS = 16
NW = NC * NS
NPAD = 10240
EPW = 10240
EPAD = EPW * NW
CHUNK = 128
NCHUNK = EPW // CHUNK
NB = 8
ROWS_PT = NPAD // NS


def _mesh():
    return plsc.VectorSubcoreMesh(core_axis_name="c", subcore_axis_name="s")


_SC_PARAMS = pltpu.CompilerParams(use_tc_tiling_on_sc=False)


@functools.partial(
    pl.kernel,
    out_type=jax.ShapeDtypeStruct((NC, NPAD), jnp.float32),
    mesh=_mesh(),
    compiler_params=_SC_PARAMS,
    scratch_types=[
        pltpu.VMEM((NCHUNK, 2, CHUNK), jnp.int32),
        pltpu.VMEM((CHUNK,), jnp.float32),
        pltpu.VMEM((ROWS_PT,), jnp.float32),
        pltpu.VMEM_SHARED((NPAD,), jnp.float32),
        pltpu.SemaphoreType.DMA,
    ],
)
def _sc_degree(ei_hbm, out_hbm, idxb, onesb, zb, degsh, sem):
    c = lax.axis_index("c")
    s = lax.axis_index("s")
    wid = c * NS + s

    @pl.loop(0, ROWS_PT // 16)
    def _(i):
        zb[pl.ds(i * 16, 16)] = jnp.zeros((16,), jnp.float32)

    @pl.loop(0, CHUNK // 16)
    def _(i):
        onesb[pl.ds(i * 16, 16)] = jnp.ones((16,), jnp.float32)

    pltpu.sync_copy(zb, degsh.at[pl.ds(s * ROWS_PT, ROWS_PT)])
    pltpu.sync_copy(ei_hbm.at[wid], idxb)
    plsc.subcore_barrier()

    @pl.loop(0, NCHUNK, step=8)
    def _(j0):
        @pl.loop(0, 8)
        def _(b):
            pltpu.async_copy(onesb, degsh.at[idxb.at[j0 + b, 1]], sem,
                             add=True)

        @pl.loop(0, 8)
        def _(b):
            pltpu.make_async_copy(onesb, degsh.at[idxb.at[0, 1]], sem).wait()

    plsc.subcore_barrier()
    pltpu.sync_copy(degsh.at[pl.ds(s * ROWS_PT, ROWS_PT)],
                    out_hbm.at[c, pl.ds(s * ROWS_PT, ROWS_PT)])


def _make_sc_agg(d):
    @functools.partial(
        pl.kernel,
        out_type=jax.ShapeDtypeStruct((NC, NPAD, d), jnp.float32),
        mesh=_mesh(),
        compiler_params=_SC_PARAMS,
        scratch_types=[
            pltpu.VMEM((NCHUNK, 2, CHUNK), jnp.int32),
            [pltpu.VMEM((CHUNK, d), jnp.float32) for _ in range(NB)],
            pltpu.VMEM((ROWS_PT, d), jnp.float32),
            pltpu.VMEM_SHARED((NPAD, d), jnp.float32),
            [pltpu.SemaphoreType.DMA for _ in range(NB)],
            [pltpu.SemaphoreType.DMA for _ in range(NB)],
        ],
    )
    def sc_agg(g_hbm, ei_hbm, out_hbm, idxb, rowsb, zb, aggsh, gsem, ssem):
        c = lax.axis_index("c")
        s = lax.axis_index("s")
        wid = c * NS + s

        @pl.loop(0, ROWS_PT)
        def _(i):
            zb[i, :] = jnp.zeros((d,), jnp.float32)

        pltpu.sync_copy(zb, aggsh.at[pl.ds(s * ROWS_PT, ROWS_PT)])
        pltpu.sync_copy(ei_hbm.at[wid], idxb)
        plsc.subcore_barrier()

        def start_gather(j, b):
            pltpu.async_copy(g_hbm.at[idxb.at[j, 0]], rowsb[b], gsem[b])

        def wait_gather(b):
            pltpu.make_async_copy(g_hbm.at[idxb.at[0, 0]], rowsb[b],
                                  gsem[b]).wait()

        def start_scatter(j, b):
            pltpu.async_copy(rowsb[b], aggsh.at[idxb.at[j, 1]], ssem[b],
                             add=True)

        def wait_scatter(b):
            pltpu.make_async_copy(rowsb[b], aggsh.at[idxb.at[0, 1]],
                                  ssem[b]).wait()

        for b in range(NB):
            start_gather(b, b)

        @pl.loop(0, (NCHUNK - NB) // NB)
        def _(gg):
            base = gg * NB
            for b in range(NB):
                j = base + b
                wait_gather(b)
                start_scatter(j, b)
                wait_scatter(b)
                start_gather(j + NB, b)

        for b in range(NB):
            j = NCHUNK - NB + b
            wait_gather(b)
            start_scatter(j, b)
            wait_scatter(b)

        plsc.subcore_barrier()
        pltpu.sync_copy(aggsh.at[pl.ds(s * ROWS_PT, ROWS_PT)],
                        out_hbm.at[c, pl.ds(s * ROWS_PT, ROWS_PT)])

    return sc_agg


_sc_agg16 = _make_sc_agg(D_HID)


def _tc_matmul_body(x_ref, w_ref, o_ref):
    o_ref[...] = jnp.dot(x_ref[...], w_ref[...],
                         preferred_element_type=jnp.float32)


_tc_matmul = pl.pallas_call(
    _tc_matmul_body,
    out_shape=jax.ShapeDtypeStruct((NPAD, D_HID), jnp.float32),
)


def _tc_scale1_body(h_ref, degp_ref, g_ref, norm_ref):
    deg = degp_ref[0, :] + degp_ref[1, :]
    norm = lax.rsqrt(jnp.maximum(deg, 1.0))
    norm_ref[...] = norm
    g_ref[...] = h_ref[...] * norm[:, None]


_tc_scale1 = pl.pallas_call(
    _tc_scale1_body,
    out_shape=[
        jax.ShapeDtypeStruct((NPAD, D_HID), jnp.float32),
        jax.ShapeDtypeStruct((NPAD,), jnp.float32),
    ],
)


def _tc_scale2_body(p_ref, norm_ref, b1_ref, w2_ref, g2_ref):
    norm = norm_ref[...]
    agg = norm[:, None] * (p_ref[0] + p_ref[1]) + b1_ref[...]
    h1 = jnp.maximum(agg, 0.0)
    g2_ref[...] = jnp.dot(h1, w2_ref[...],
                          preferred_element_type=jnp.float32) * norm[:, None]


_tc_scale2 = pl.pallas_call(
    _tc_scale2_body,
    out_shape=jax.ShapeDtypeStruct((NPAD, D_HID), jnp.float32),
)


def _tc_scale3_body(p_ref, norm_ref, b2_ref, o_ref):
    o_ref[...] = norm_ref[...][:, None] * (p_ref[0] + p_ref[1]) + b2_ref[...]


_tc_scale3 = pl.pallas_call(
    _tc_scale3_body,
    out_shape=jax.ShapeDtypeStruct((NPAD, D_HID), jnp.float32),
)


def kernel(x, edge_index, W1, b1, W2, b2):
    x_pad = jnp.zeros((NPAD, D_FEAT), jnp.float32).at[:N_NODES].set(x)
    pad = jnp.full((EPAD - N_EDGES,), N_NODES, jnp.int32)
    src = jnp.concatenate([edge_index[0], pad]).reshape(NW, NCHUNK, 1, CHUNK)
    dst = jnp.concatenate([edge_index[1], pad]).reshape(NW, NCHUNK, 1, CHUNK)
    ei = jnp.concatenate([src, dst], axis=2)
    w2p = jnp.zeros((D_HID, D_HID), jnp.float32).at[:, :N_CLASSES].set(W2)
    b1r = b1.reshape(1, D_HID)
    b2p = jnp.zeros((1, D_HID), jnp.float32).at[0, :N_CLASSES].set(b2)

    h = _tc_matmul(x_pad, W1)
    degp = _sc_degree(ei)
    g1, norm = _tc_scale1(h, degp)
    p1 = _sc_agg16(g1, ei)
    g2 = _tc_scale2(p1, norm, b1r, w2p)
    p2 = _sc_agg16(g2, ei)
    out = _tc_scale3(p2, norm, b2p)
    return out[:N_NODES, :N_CLASSES]


# speedup vs baseline: 1.4785x; 1.4785x over previous
import functools

import jax
import jax.numpy as jnp
from jax import lax
from jax.experimental import pallas as pl
from jax.experimental.pallas import tpu as pltpu
from jax.experimental.pallas import tpu_sc as plsc

N_NODES = 10000
N_EDGES = 320000
D_FEAT = 128
D_HID = 16
N_CLASSES = 7

NC = 2
NS = 16
NW = NC * NS
NPAD = 10240
EPW = 10240
EPAD = EPW * NW
CHUNK = 128
NCHUNK = EPW // CHUNK
NB = 8
ROWS_PT = NPAD // NS


def _mesh():
    return plsc.VectorSubcoreMesh(core_axis_name="c", subcore_axis_name="s")


_SC_PARAMS = pltpu.CompilerParams(use_tc_tiling_on_sc=False)


@functools.partial(
    pl.kernel,
    out_type=jax.ShapeDtypeStruct((NC, NPAD), jnp.float32),
    mesh=_mesh(),
    compiler_params=_SC_PARAMS,
    scratch_types=[
        pltpu.VMEM((NCHUNK, 2, CHUNK), jnp.int32),
        pltpu.VMEM((CHUNK,), jnp.float32),
        pltpu.VMEM((ROWS_PT,), jnp.float32),
        pltpu.VMEM_SHARED((NPAD,), jnp.float32),
        pltpu.SemaphoreType.DMA,
    ],
)
def _sc_degree(ei_hbm, out_hbm, idxb, onesb, zb, degsh, sem):
    c = lax.axis_index("c")
    s = lax.axis_index("s")
    wid = c * NS + s

    @pl.loop(0, ROWS_PT // 16)
    def _(i):
        zb[pl.ds(i * 16, 16)] = jnp.zeros((16,), jnp.float32)

    @pl.loop(0, CHUNK // 16)
    def _(i):
        onesb[pl.ds(i * 16, 16)] = jnp.ones((16,), jnp.float32)

    pltpu.sync_copy(zb, degsh.at[pl.ds(s * ROWS_PT, ROWS_PT)])
    pltpu.sync_copy(ei_hbm.at[wid], idxb)
    plsc.subcore_barrier()

    @pl.loop(0, NCHUNK, step=8)
    def _(j0):
        @pl.loop(0, 8)
        def _(b):
            pltpu.async_copy(onesb, degsh.at[idxb.at[j0 + b, 1]], sem,
                             add=True)

        @pl.loop(0, 8)
        def _(b):
            pltpu.make_async_copy(onesb, degsh.at[idxb.at[0, 1]], sem).wait()

    plsc.subcore_barrier()
    pltpu.sync_copy(degsh.at[pl.ds(s * ROWS_PT, ROWS_PT)],
                    out_hbm.at[c, pl.ds(s * ROWS_PT, ROWS_PT)])


def _make_sc_agg(d):
    @functools.partial(
        pl.kernel,
        out_type=jax.ShapeDtypeStruct((NC, NPAD, d), jnp.float32),
        mesh=_mesh(),
        compiler_params=_SC_PARAMS,
        scratch_types=[
            pltpu.VMEM((NCHUNK, 2, CHUNK), jnp.int32),
            [pltpu.VMEM((CHUNK, d), jnp.float32) for _ in range(NB)],
            pltpu.VMEM((ROWS_PT, d), jnp.float32),
            pltpu.VMEM_SHARED((NPAD, d), jnp.float32),
            pltpu.VMEM_SHARED((NPAD, d), jnp.float32),
            [pltpu.SemaphoreType.DMA for _ in range(NB)],
            [pltpu.SemaphoreType.DMA for _ in range(NB)],
        ],
    )
    def sc_agg(g_hbm, ei_hbm, out_hbm, idxb, rowsb, zb, aggsh, gsh, gsem, ssem):
        c = lax.axis_index("c")
        s = lax.axis_index("s")
        wid = c * NS + s

        @pl.loop(0, ROWS_PT)
        def _(i):
            zb[i, :] = jnp.zeros((d,), jnp.float32)

        pltpu.sync_copy(zb, aggsh.at[pl.ds(s * ROWS_PT, ROWS_PT)])
        pltpu.sync_copy(g_hbm.at[pl.ds(s * ROWS_PT, ROWS_PT)],
                        gsh.at[pl.ds(s * ROWS_PT, ROWS_PT)])
        pltpu.sync_copy(ei_hbm.at[wid], idxb)
        plsc.subcore_barrier()

        def start_gather(j, b):
            pltpu.async_copy(gsh.at[idxb.at[j, 0]], rowsb[b], gsem[b])

        def wait_gather(b):
            pltpu.make_async_copy(gsh.at[idxb.at[0, 0]], rowsb[b],
                                  gsem[b]).wait()

        def start_scatter(j, b):
            pltpu.async_copy(rowsb[b], aggsh.at[idxb.at[j, 1]], ssem[b],
                             add=True)

        def wait_scatter(b):
            pltpu.make_async_copy(rowsb[b], aggsh.at[idxb.at[0, 1]],
                                  ssem[b]).wait()

        for b in range(NB):
            start_gather(b, b)

        @pl.loop(0, (NCHUNK - NB) // NB)
        def _(gg):
            base = gg * NB
            for b in range(NB):
                j = base + b
                wait_gather(b)
                start_scatter(j, b)
                wait_scatter(b)
                start_gather(j + NB, b)

        for b in range(NB):
            j = NCHUNK - NB + b
            wait_gather(b)
            start_scatter(j, b)
            wait_scatter(b)

        plsc.subcore_barrier()
        pltpu.sync_copy(aggsh.at[pl.ds(s * ROWS_PT, ROWS_PT)],
                        out_hbm.at[c, pl.ds(s * ROWS_PT, ROWS_PT)])

    return sc_agg


_sc_agg16 = _make_sc_agg(D_HID)


def _tc_matmul_body(x_ref, w_ref, o_ref):
    o_ref[...] = jnp.dot(x_ref[...], w_ref[...],
                         preferred_element_type=jnp.float32)


_tc_matmul = pl.pallas_call(
    _tc_matmul_body,
    out_shape=jax.ShapeDtypeStruct((NPAD, D_HID), jnp.float32),
)


def _tc_scale1_body(h_ref, degp_ref, g_ref, norm_ref):
    deg = degp_ref[0, :] + degp_ref[1, :]
    norm = lax.rsqrt(jnp.maximum(deg, 1.0))
    norm_ref[...] = norm
    g_ref[...] = h_ref[...] * norm[:, None]


_tc_scale1 = pl.pallas_call(
    _tc_scale1_body,
    out_shape=[
        jax.ShapeDtypeStruct((NPAD, D_HID), jnp.float32),
        jax.ShapeDtypeStruct((NPAD,), jnp.float32),
    ],
)


def _tc_scale2_body(p_ref, norm_ref, b1_ref, w2_ref, g2_ref):
    norm = norm_ref[...]
    agg = norm[:, None] * (p_ref[0] + p_ref[1]) + b1_ref[...]
    h1 = jnp.maximum(agg, 0.0)
    g2_ref[...] = jnp.dot(h1, w2_ref[...],
                          preferred_element_type=jnp.float32) * norm[:, None]


_tc_scale2 = pl.pallas_call(
    _tc_scale2_body,
    out_shape=jax.ShapeDtypeStruct((NPAD, D_HID), jnp.float32),
)


def _tc_scale3_body(p_ref, norm_ref, b2_ref, o_ref):
    o_ref[...] = norm_ref[...][:, None] * (p_ref[0] + p_ref[1]) + b2_ref[...]


_tc_scale3 = pl.pallas_call(
    _tc_scale3_body,
    out_shape=jax.ShapeDtypeStruct((NPAD, D_HID), jnp.float32),
)


def kernel(x, edge_index, W1, b1, W2, b2):
    x_pad = jnp.zeros((NPAD, D_FEAT), jnp.float32).at[:N_NODES].set(x)
    pad = jnp.full((EPAD - N_EDGES,), N_NODES, jnp.int32)
    src = jnp.concatenate([edge_index[0], pad]).reshape(NW, NCHUNK, 1, CHUNK)
    dst = jnp.concatenate([edge_index[1], pad]).reshape(NW, NCHUNK, 1, CHUNK)
    ei = jnp.concatenate([src, dst], axis=2)
    w2p = jnp.zeros((D_HID, D_HID), jnp.float32).at[:, :N_CLASSES].set(W2)
    b1r = b1.reshape(1, D_HID)
    b2p = jnp.zeros((1, D_HID), jnp.float32).at[0, :N_CLASSES].set(b2)

    h = _tc_matmul(x_pad, W1)
    degp = _sc_degree(ei)
    g1, norm = _tc_scale1(h, degp)
    p1 = _sc_agg16(g1, ei)
    g2 = _tc_scale2(p1, norm, b1r, w2p)
    p2 = _sc_agg16(g2, ei)
    out = _tc_scale3(p2, norm, b2p)
    return out[:N_NODES, :N_CLASSES]
